# baseline (speedup 1.0000x reference)
import dataclasses
import functools
import jax
import jax.numpy as jnp
from jax.experimental import pallas as pl
from jax.experimental.pallas import tpu as pltpu
from jax.experimental.pallas import tpu_sc as plsc

N_NODES = 10000
N_EDGES = 160000
N_PAD2 = 10240
ACC_ROWS = N_PAD2 + 8
G = 128
E_PAD2 = 163840
BLOCKS = E_PAD2 // G
BR = 400


def _sc_segsum(values2d, srcs, dst_e, zeros_blk, nchunks, split):
    out_chunks = nchunks if not split else 2
    drain = N_PAD2 // 16

    mesh = plsc.VectorSubcoreMesh(core_axis_name="c", subcore_axis_name="s")
    cp = pltpu.CompilerParams()
    if "needs_layout_passes" in pltpu.CompilerParams.__dataclass_fields__:
        cp = dataclasses.replace(cp, needs_layout_passes=False)

    @functools.partial(
        pl.kernel,
        out_type=jax.ShapeDtypeStruct((out_chunks * N_PAD2, 128), jnp.float32),
        mesh=mesh,
        compiler_params=cp,
        scratch_types=[
            pltpu.VMEM_SHARED((ACC_ROWS, 128), jnp.float32),
            pltpu.VMEM((2, G, 128), jnp.float32),
            pltpu.VMEM((4, G), jnp.int32),
            pltpu.VMEM((4, G), jnp.int32),
            pltpu.SemaphoreType.DMA((4,)),
            pltpu.SemaphoreType.DMA((2,)),
            pltpu.SemaphoreType.DMA((2,)),
        ],
    )
    def k(*refs):
        vals_hbm = refs[0]
        src_hbms = refs[1:1 + nchunks]
        dst_hbm = refs[1 + nchunks]
        zero_hbm = refs[2 + nchunks]
        out_hbm = refs[3 + nchunks]
        acc, gbuf, sidx, didx, sem_i, sem_g, sem_s = refs[4 + nchunks:]
        core = jax.lax.axis_index("c")
        sid = jax.lax.axis_index("s")

        def run_chunk(src_hbm, out_base, blk_base, trips):
            pltpu.sync_copy(zero_hbm, acc.at[pl.ds(sid * drain, drain)])

            @pl.when(sid == 0)
            def _():
                pltpu.sync_copy(zero_hbm.at[pl.ds(0, 8)],
                                acc.at[pl.ds(N_PAD2, 8)])
            plsc.subcore_barrier()

            def fire_idx(j, slot):
                e = (blk_base + j) * G
                pltpu.async_copy(src_hbm.at[pl.ds(e, G)], sidx.at[slot],
                                 sem_i.at[slot])
                pltpu.async_copy(dst_hbm.at[pl.ds(e, G)], didx.at[slot],
                                 sem_i.at[slot])

            def wait_idx(slot):
                pltpu.make_async_copy(src_hbm.at[pl.ds(0, G)], sidx.at[slot],
                                      sem_i.at[slot]).wait()
                pltpu.make_async_copy(dst_hbm.at[pl.ds(0, G)], didx.at[slot],
                                      sem_i.at[slot]).wait()

            def fire_gather(slot, b2):
                pltpu.async_copy(vals_hbm.at[sidx.at[slot]], gbuf.at[b2],
                                 sem_g.at[b2])

            def wait_gather(slot, b2):
                pltpu.make_async_copy(vals_hbm.at[sidx.at[slot]],
                                      gbuf.at[b2], sem_g.at[b2]).wait()

            def fire_scat(slot, b2):
                pltpu.async_copy(gbuf.at[b2], acc.at[didx.at[slot]],
                                 sem_s.at[b2], add=True)

            def wait_scat(slot, b2):
                pltpu.make_async_copy(gbuf.at[b2], acc.at[didx.at[slot]],
                                      sem_s.at[b2]).wait()

            def step(j, u):
                wait_idx(u)
                fire_gather(u, u % 2)
                fire_idx(j + 2, (u + 2) % 4)

            fire_idx(0, 0)
            fire_idx(1, 1)
            step(0, 0)
            step(1, 1)
            wait_gather(0, 0)
            fire_scat(0, 0)
            wait_scat(0, 0)
            step(2, 2)
            wait_gather(1, 1)
            fire_scat(1, 1)
            wait_scat(1, 1)
            step(3, 3)
            wait_gather(2, 0)
            fire_scat(2, 0)

            @pl.loop(1, trips // 4 - 1)
            def _quad(jo):
                for u in range(4):
                    j = jo * 4 + u
                    b2 = u % 2
                    wait_scat((u + 2) % 4, b2)
                    wait_idx(u)
                    fire_gather(u, b2)
                    fire_idx(j + 2, (u + 2) % 4)
                    wait_gather((u + 3) % 4, 1 - b2)
                    fire_scat((u + 3) % 4, 1 - b2)

            tl = trips - 4
            for u in range(4):
                b2 = u % 2
                wait_scat((u + 2) % 4, b2)
                wait_idx(u)
                fire_gather(u, b2)
                if u < 2:
                    fire_idx(tl + u + 2, (u + 2) % 4)
                wait_gather((u + 3) % 4, 1 - b2)
                fire_scat((u + 3) % 4, 1 - b2)
            wait_gather(3, 1)
            fire_scat(3, 1)
            wait_scat(2, 0)
            wait_scat(3, 1)

            plsc.subcore_barrier()
            pltpu.sync_copy(
                acc.at[pl.ds(sid * drain, drain)],
                out_hbm.at[pl.ds(out_base + sid * drain, drain)],
            )
            plsc.subcore_barrier()

        if split:
            run_chunk(src_hbms[0], core * N_PAD2,
                      core * (BLOCKS // 2) + sid * (BLOCKS // 32),
                      BLOCKS // 32)
        else:
            for i in range(nchunks // 2):
                for cv in (0, 1):
                    c = 2 * i + cv

                    @pl.when(core == cv)
                    def _(c=c):
                        run_chunk(src_hbms[c], c * N_PAD2,
                                  sid * (BLOCKS // 16), BLOCKS // 16)

    return k(values2d, *srcs, dst_e, zeros_blk)


def _layer_tc(aggs, xin, Wl, bl, Wr, cnt0, cnt1, want_relu, two_out):
    n, cin = xin.shape
    cout = Wl.shape[1]
    grid = n // BR
    k = len(aggs)

    def body(*refs):
        agg_refs = refs[:k]
        x_ref, wl_ref, bl_ref, wr_ref, c0_ref, c1_ref = refs[k:k + 6]
        outs = refs[k + 6:]
        cnt = (c0_ref[:, 0:1] + c1_ref[:, 0:1])
        inv = 1.0 / jnp.maximum(cnt, 1.0)
        a = jnp.concatenate([r[...] for r in agg_refs], axis=1) * inv
        out = (jnp.dot(a, wl_ref[...], preferred_element_type=jnp.float32)
               + bl_ref[...]
               + jnp.dot(x_ref[...], wr_ref[...],
                         preferred_element_type=jnp.float32))
        if two_out:
            outs[0][...] = out
            outs[1][...] = jnp.maximum(out, 0.0)
        elif want_relu:
            outs[0][...] = jnp.maximum(out, 0.0)
        else:
            outs[0][...] = out

    out_shape = jax.ShapeDtypeStruct((n, cout), jnp.float32)
    out_shapes = (out_shape, out_shape) if two_out else out_shape
    out_spec = pl.BlockSpec((BR, cout), lambda i: (i, 0))
    out_specs = (out_spec, out_spec) if two_out else out_spec
    chunk_spec = pl.BlockSpec((BR, 128), lambda i: (i, 0))

    return pl.pallas_call(
        body,
        grid=(grid,),
        in_specs=(
            [chunk_spec] * k
            + [
                pl.BlockSpec((BR, cin), lambda i: (i, 0)),
                pl.BlockSpec((cin, cout), lambda i: (0, 0)),
                pl.BlockSpec((1, cout), lambda i: (0, 0)),
                pl.BlockSpec((cin, cout), lambda i: (0, 0)),
                chunk_spec,
                chunk_spec,
            ]
        ),
        out_specs=out_specs,
        out_shape=out_shapes,
    )(*aggs, xin, Wl, bl, Wr, cnt0, cnt1)


def _mm2_tc(h2, Wl2p, Wr2p):
    n, cin = h2.shape
    cout = Wl2p.shape[1]
    grid = n // BR

    def body(h_ref, wl_ref, wr_ref, y_ref, z_ref):
        h = h_ref[...]
        y_ref[...] = jnp.dot(h, wl_ref[...], preferred_element_type=jnp.float32)
        z_ref[...] = jnp.dot(h, wr_ref[...], preferred_element_type=jnp.float32)

    shp = jax.ShapeDtypeStruct((n, cout), jnp.float32)
    spec = pl.BlockSpec((BR, cout), lambda i: (i, 0))
    return pl.pallas_call(
        body,
        grid=(grid,),
        in_specs=[
            pl.BlockSpec((BR, cin), lambda i: (i, 0)),
            pl.BlockSpec((cin, cout), lambda i: (0, 0)),
            pl.BlockSpec((cin, cout), lambda i: (0, 0)),
        ],
        out_specs=(spec, spec),
        out_shape=(shp, shp),
    )(h2, Wl2p, Wr2p)


def _final_tc(a2a, a2b, z2, bl2p, cnt0, cnt1, valid):
    n, cout = z2.shape
    grid = n // BR

    def body(a_ref, b_ref, z_ref, bl_ref, c0_ref, c1_ref,
             logits_ref, logp_ref):
        cnt = (c0_ref[:, 0:1] + c1_ref[:, 0:1])
        inv = 1.0 / jnp.maximum(cnt, 1.0)
        logits = (a_ref[...] + b_ref[...]) * inv + bl_ref[...] + z_ref[...]
        logits_ref[...] = logits
        col = jax.lax.broadcasted_iota(jnp.int32, (BR, cout), 1)
        mask = col < valid
        neg = jnp.float32(-1e30)
        lm = jnp.where(mask, logits, neg)
        mx = jnp.max(lm, axis=1, keepdims=True)
        ex = jnp.where(mask, jnp.exp(logits - mx), 0.0)
        lse = jnp.log(jnp.sum(ex, axis=1, keepdims=True))
        logp_ref[...] = logits - mx - lse

    shp = jax.ShapeDtypeStruct((n, cout), jnp.float32)
    spec = pl.BlockSpec((BR, cout), lambda i: (i, 0))
    return pl.pallas_call(
        body,
        grid=(grid,),
        in_specs=[spec, spec, spec,
                  pl.BlockSpec((1, cout), lambda i: (0, 0)),
                  spec, spec],
        out_specs=(spec, spec),
        out_shape=(shp, shp),
    )(a2a, a2b, z2, bl2p, cnt0, cnt1)


def kernel(x, edge_index, Wl0, bl0, Wr0, Wl1, bl1, Wr1, Wl2, bl2, Wr2):
    src = edge_index[0]
    dst = edge_index[1]

    extra = E_PAD2 - N_EDGES
    dst_p = jnp.concatenate([dst, jnp.full((extra,), N_PAD2, jnp.int32)])
    src_p = jnp.concatenate([src, jnp.zeros((extra,), jnp.int32)])
    srcs2 = [src_p * 2 + c for c in range(2)]
    srcs4 = [src_p * 4 + c for c in range(4)]
    zero_idx = jnp.zeros((E_PAD2,), jnp.int32)
    zeros_blk = jnp.zeros((N_PAD2 // 16, 128), jnp.float32)
    ones2d = jnp.ones((8, 128), jnp.float32)

    bl0r = bl0.reshape(1, -1)
    bl1r = bl1.reshape(1, -1)
    Wl2p = jnp.pad(Wl2, ((0, 0), (0, 128 - 47)))
    Wr2p = jnp.pad(Wr2, ((0, 0), (0, 128 - 47)))
    bl2p = jnp.pad(bl2, (0, 128 - 47)).reshape(1, -1)

    cnts = _sc_segsum(ones2d, [zero_idx], dst_p, zeros_blk, 1, split=True)
    cnt0 = cnts[0:N_NODES]
    cnt1 = cnts[N_PAD2:N_PAD2 + N_NODES]

    agg0 = _sc_segsum(x.reshape(-1, 128), srcs2, dst_p, zeros_blk, 2,
                      split=False)
    a00 = agg0[0:N_NODES]
    a01 = agg0[N_PAD2:N_PAD2 + N_NODES]
    h = _layer_tc([a00, a01], x, Wl0, bl0r, Wr0, cnt0, cnt1,
                  want_relu=True, two_out=False)

    agg1 = _sc_segsum(h.reshape(-1, 128), srcs4, dst_p, zeros_blk, 4,
                      split=False)
    a1 = [agg1[c * N_PAD2:c * N_PAD2 + N_NODES] for c in range(4)]
    out2, h2 = _layer_tc(a1, h, Wl1, bl1r, Wr1, cnt0, cnt1,
                         want_relu=True, two_out=True)

    y2, z2row = _mm2_tc(h2, Wl2p, Wr2p)
    agg2 = _sc_segsum(y2, [src_p], dst_p, zeros_blk, 1, split=True)
    a2a = agg2[0:N_NODES]
    a2b = agg2[N_PAD2:N_PAD2 + N_NODES]
    logits_p, logp_p = _final_tc(a2a, a2b, z2row, bl2p, cnt0, cnt1, 47)

    logits = logits_p[:, :47]
    logp = logp_p[:, :47]
    return (logp, out2, h2, logits)


# speedup vs baseline: 4.5730x; 4.5730x over previous
import dataclasses
import functools
import jax
import jax.numpy as jnp
from jax.experimental import pallas as pl
from jax.experimental.pallas import tpu as pltpu
from jax.experimental.pallas import tpu_sc as plsc

N_NODES = 10000
N_EDGES = 160000
N_PAD2 = 10240
ACC_ROWS = N_PAD2 + 8
G = 128
E_PAD2 = 163840
BLOCKS = E_PAD2 // G
BR = 400


def _sc_segsum(values2d, srcs, dst_e, zeros_blk, nchunks, split):
    out_chunks = nchunks if not split else 2
    drain = N_PAD2 // 16

    mesh = plsc.VectorSubcoreMesh(core_axis_name="c", subcore_axis_name="s")
    cp = pltpu.CompilerParams()
    if "needs_layout_passes" in pltpu.CompilerParams.__dataclass_fields__:
        cp = dataclasses.replace(cp, needs_layout_passes=False)

    @functools.partial(
        pl.kernel,
        out_type=jax.ShapeDtypeStruct((out_chunks * N_PAD2, 128), jnp.float32),
        mesh=mesh,
        compiler_params=cp,
        scratch_types=[
            pltpu.VMEM_SHARED((ACC_ROWS, 128), jnp.float32),
            pltpu.VMEM((2, G, 128), jnp.float32),
            pltpu.VMEM((4, G), jnp.int32),
            pltpu.VMEM((4, G), jnp.int32),
            pltpu.SemaphoreType.DMA((4,)),
            pltpu.SemaphoreType.DMA((2,)),
            pltpu.SemaphoreType.DMA((2,)),
        ],
    )
    def k(*refs):
        vals_hbm = refs[0]
        src_hbms = refs[1:1 + nchunks]
        dst_hbm = refs[1 + nchunks]
        zero_hbm = refs[2 + nchunks]
        out_hbm = refs[3 + nchunks]
        acc, gbuf, sidx, didx, sem_i, sem_g, sem_s = refs[4 + nchunks:]
        core = jax.lax.axis_index("c")
        sid = jax.lax.axis_index("s")

        def run_chunk(src_hbm, out_base, blk_base, trips):
            pltpu.sync_copy(zero_hbm, acc.at[pl.ds(sid * drain, drain)])

            @pl.when(sid == 0)
            def _():
                pltpu.sync_copy(zero_hbm.at[pl.ds(0, 8)],
                                acc.at[pl.ds(N_PAD2, 8)])
            plsc.subcore_barrier()

            def fire_idx(j, slot):
                e = (blk_base + j) * G
                pltpu.async_copy(src_hbm.at[pl.ds(e, G)], sidx.at[slot],
                                 sem_i.at[slot])
                pltpu.async_copy(dst_hbm.at[pl.ds(e, G)], didx.at[slot],
                                 sem_i.at[slot])

            def wait_idx(slot):
                pltpu.make_async_copy(src_hbm.at[pl.ds(0, G)], sidx.at[slot],
                                      sem_i.at[slot]).wait()
                pltpu.make_async_copy(dst_hbm.at[pl.ds(0, G)], didx.at[slot],
                                      sem_i.at[slot]).wait()

            def fire_gather(slot, b2):
                pltpu.async_copy(vals_hbm.at[sidx.at[slot]], gbuf.at[b2],
                                 sem_g.at[b2])

            def wait_gather(slot, b2):
                pltpu.make_async_copy(vals_hbm.at[sidx.at[slot]],
                                      gbuf.at[b2], sem_g.at[b2]).wait()

            def fire_scat(slot, b2):
                pltpu.async_copy(gbuf.at[b2], acc.at[didx.at[slot]],
                                 sem_s.at[b2], add=True)

            def wait_scat(slot, b2):
                pltpu.make_async_copy(gbuf.at[b2], acc.at[didx.at[slot]],
                                      sem_s.at[b2]).wait()

            def step(j, u):
                wait_idx(u)
                fire_gather(u, u % 2)
                fire_idx(j + 2, (u + 2) % 4)

            fire_idx(0, 0)
            fire_idx(1, 1)
            step(0, 0)
            step(1, 1)
            wait_gather(0, 0)
            fire_scat(0, 0)
            wait_scat(0, 0)
            step(2, 2)
            wait_gather(1, 1)
            fire_scat(1, 1)
            wait_scat(1, 1)
            step(3, 3)
            wait_gather(2, 0)
            fire_scat(2, 0)

            @pl.loop(1, trips // 4 - 1)
            def _quad(jo):
                for u in range(4):
                    j = jo * 4 + u
                    b2 = u % 2
                    wait_scat((u + 2) % 4, b2)
                    wait_idx(u)
                    fire_gather(u, b2)
                    fire_idx(j + 2, (u + 2) % 4)
                    wait_gather((u + 3) % 4, 1 - b2)
                    fire_scat((u + 3) % 4, 1 - b2)

            tl = trips - 4
            for u in range(4):
                b2 = u % 2
                wait_scat((u + 2) % 4, b2)
                wait_idx(u)
                fire_gather(u, b2)
                if u < 2:
                    fire_idx(tl + u + 2, (u + 2) % 4)
                wait_gather((u + 3) % 4, 1 - b2)
                fire_scat((u + 3) % 4, 1 - b2)
            wait_gather(3, 1)
            fire_scat(3, 1)
            wait_scat(2, 0)
            wait_scat(3, 1)

            plsc.subcore_barrier()
            pltpu.sync_copy(
                acc.at[pl.ds(sid * drain, drain)],
                out_hbm.at[pl.ds(out_base + sid * drain, drain)],
            )
            plsc.subcore_barrier()

        if split:
            run_chunk(src_hbms[0], core * N_PAD2,
                      core * (BLOCKS // 2) + sid * (BLOCKS // 32),
                      BLOCKS // 32)
        else:
            for i in range(nchunks // 2):
                for cv in (0, 1):
                    c = 2 * i + cv

                    @pl.when(core == cv)
                    def _(c=c):
                        run_chunk(src_hbms[c], c * N_PAD2,
                                  sid * (BLOCKS // 16), BLOCKS // 16)

    return k(values2d, *srcs, dst_e, zeros_blk)


def _layer_tc(aggs, xin, Wl, bl, Wr, cnt0, cnt1, want_relu, two_out):
    n, cin = xin.shape
    cout = Wl.shape[1]
    grid = n // BR
    k = len(aggs)

    def body(*refs):
        agg_refs = refs[:k]
        x_ref, wl_ref, bl_ref, wr_ref, c0_ref, c1_ref = refs[k:k + 6]
        outs = refs[k + 6:]
        cnt = (c0_ref[:, 0:1] + c1_ref[:, 0:1])
        inv = 1.0 / jnp.maximum(cnt, 1.0)
        a = jnp.concatenate([r[...] for r in agg_refs], axis=1) * inv
        out = (jnp.dot(a, wl_ref[...], preferred_element_type=jnp.float32)
               + bl_ref[...]
               + jnp.dot(x_ref[...], wr_ref[...],
                         preferred_element_type=jnp.float32))
        if two_out:
            outs[0][...] = out
            outs[1][...] = jnp.maximum(out, 0.0)
        elif want_relu:
            outs[0][...] = jnp.maximum(out, 0.0)
        else:
            outs[0][...] = out

    out_shape = jax.ShapeDtypeStruct((n, cout), jnp.float32)
    out_shapes = (out_shape, out_shape) if two_out else out_shape
    out_spec = pl.BlockSpec((BR, cout), lambda i: (i, 0))
    out_specs = (out_spec, out_spec) if two_out else out_spec
    chunk_spec = pl.BlockSpec((BR, 128), lambda i: (i, 0))

    return pl.pallas_call(
        body,
        grid=(grid,),
        in_specs=(
            [chunk_spec] * k
            + [
                pl.BlockSpec((BR, cin), lambda i: (i, 0)),
                pl.BlockSpec((cin, cout), lambda i: (0, 0)),
                pl.BlockSpec((1, cout), lambda i: (0, 0)),
                pl.BlockSpec((cin, cout), lambda i: (0, 0)),
                chunk_spec,
                chunk_spec,
            ]
        ),
        out_specs=out_specs,
        out_shape=out_shapes,
    )(*aggs, xin, Wl, bl, Wr, cnt0, cnt1)


def _mm2_tc(h2, Wl2p, Wr2p):
    n, cin = h2.shape
    cout = Wl2p.shape[1]
    grid = n // BR

    def body(h_ref, wl_ref, wr_ref, y_ref, z_ref):
        h = h_ref[...]
        y_ref[...] = jnp.dot(h, wl_ref[...], preferred_element_type=jnp.float32)
        z_ref[...] = jnp.dot(h, wr_ref[...], preferred_element_type=jnp.float32)

    shp = jax.ShapeDtypeStruct((n, cout), jnp.float32)
    spec = pl.BlockSpec((BR, cout), lambda i: (i, 0))
    return pl.pallas_call(
        body,
        grid=(grid,),
        in_specs=[
            pl.BlockSpec((BR, cin), lambda i: (i, 0)),
            pl.BlockSpec((cin, cout), lambda i: (0, 0)),
            pl.BlockSpec((cin, cout), lambda i: (0, 0)),
        ],
        out_specs=(spec, spec),
        out_shape=(shp, shp),
    )(h2, Wl2p, Wr2p)


def _final_tc(a2a, a2b, z2, bl2p, cnt0, cnt1, valid):
    n, cout = z2.shape
    grid = n // BR

    def body(a_ref, b_ref, z_ref, bl_ref, c0_ref, c1_ref,
             logits_ref, logp_ref):
        cnt = (c0_ref[:, 0:1] + c1_ref[:, 0:1])
        inv = 1.0 / jnp.maximum(cnt, 1.0)
        logits = (a_ref[...] + b_ref[...]) * inv + bl_ref[...] + z_ref[...]
        logits_ref[...] = logits
        col = jax.lax.broadcasted_iota(jnp.int32, (BR, cout), 1)
        mask = col < valid
        neg = jnp.float32(-1e30)
        lm = jnp.where(mask, logits, neg)
        mx = jnp.max(lm, axis=1, keepdims=True)
        ex = jnp.where(mask, jnp.exp(logits - mx), 0.0)
        lse = jnp.log(jnp.sum(ex, axis=1, keepdims=True))
        logp_ref[...] = logits - mx - lse

    shp = jax.ShapeDtypeStruct((n, cout), jnp.float32)
    spec = pl.BlockSpec((BR, cout), lambda i: (i, 0))
    return pl.pallas_call(
        body,
        grid=(grid,),
        in_specs=[spec, spec, spec,
                  pl.BlockSpec((1, cout), lambda i: (0, 0)),
                  spec, spec],
        out_specs=(spec, spec),
        out_shape=(shp, shp),
    )(a2a, a2b, z2, bl2p, cnt0, cnt1)


def kernel(x, edge_index, Wl0, bl0, Wr0, Wl1, bl1, Wr1, Wl2, bl2, Wr2):
    src = edge_index[0]
    dst = edge_index[1]

    extra = E_PAD2 - N_EDGES
    dst_p = jnp.concatenate([dst, jnp.full((extra,), N_PAD2, jnp.int32)])
    src_p = jnp.concatenate([src, jnp.zeros((extra,), jnp.int32)])
    srcs2 = [src_p * 2 + c for c in range(2)]
    srcs4 = [src_p * 4 + c for c in range(4)]
    ones_idx = jnp.bitwise_and(src_p, 1023)
    zeros_blk = jnp.zeros((N_PAD2 // 16, 128), jnp.float32)
    ones2d = jnp.ones((1024, 128), jnp.float32)

    bl0r = bl0.reshape(1, -1)
    bl1r = bl1.reshape(1, -1)
    Wl2p = jnp.pad(Wl2, ((0, 0), (0, 128 - 47)))
    Wr2p = jnp.pad(Wr2, ((0, 0), (0, 128 - 47)))
    bl2p = jnp.pad(bl2, (0, 128 - 47)).reshape(1, -1)

    cnts = _sc_segsum(ones2d, [ones_idx], dst_p, zeros_blk, 1, split=True)
    cnt0 = cnts[0:N_NODES]
    cnt1 = cnts[N_PAD2:N_PAD2 + N_NODES]

    agg0 = _sc_segsum(x.reshape(-1, 128), srcs2, dst_p, zeros_blk, 2,
                      split=False)
    a00 = agg0[0:N_NODES]
    a01 = agg0[N_PAD2:N_PAD2 + N_NODES]
    h = _layer_tc([a00, a01], x, Wl0, bl0r, Wr0, cnt0, cnt1,
                  want_relu=True, two_out=False)

    agg1 = _sc_segsum(h.reshape(-1, 128), srcs4, dst_p, zeros_blk, 4,
                      split=False)
    a1 = [agg1[c * N_PAD2:c * N_PAD2 + N_NODES] for c in range(4)]
    out2, h2 = _layer_tc(a1, h, Wl1, bl1r, Wr1, cnt0, cnt1,
                         want_relu=True, two_out=True)

    y2, z2row = _mm2_tc(h2, Wl2p, Wr2p)
    agg2 = _sc_segsum(y2, [src_p], dst_p, zeros_blk, 1, split=True)
    a2a = agg2[0:N_NODES]
    a2b = agg2[N_PAD2:N_PAD2 + N_NODES]
    logits_p, logp_p = _final_tc(a2a, a2b, z2row, bl2p, cnt0, cnt1, 47)

    logits = logits_p[:, :47]
    logp = logp_p[:, :47]
    return (logp, out2, h2, logits)
